# baseline (speedup 1.0000x reference)
import functools

import jax
import jax.numpy as jnp
from jax import lax
from jax.experimental import pallas as pl
from jax.experimental.pallas import tpu as pltpu
from jax.experimental.pallas import tpu_sc as plsc

KNN_K = 3
N_IN = 12500
N_SKIP = 50000
D = 128

NIN_PAD = 12544
NW = 32
NB = 416
S1B = 256
S2B = 160
NB_OUT = 391
CHUNK = 128


def _knn_body(skip_pos_ref, in_pos_t_ref, idx_ref, w_ref):
    BIGI = jnp.int32(2**30)
    BIGD = jnp.float32(1e30)
    BY = 128
    y = skip_pos_ref[...]
    t0 = y[:, 0:1] - in_pos_t_ref[0:1, :]
    t1 = y[:, 1:2] - in_pos_t_ref[1:2, :]
    t2 = y[:, 2:3] - in_pos_t_ref[2:3, :]
    d2 = (t0 * t0 + t1 * t1) + t2 * t2
    iota = lax.broadcasted_iota(jnp.int32, (BY, NIN_PAD), 1)

    m1 = jnp.min(d2, axis=1, keepdims=True)
    i1 = jnp.min(jnp.where(d2 == m1, iota, BIGI), axis=1, keepdims=True)
    d2 = jnp.where(iota == i1, BIGD, d2)
    m2 = jnp.min(d2, axis=1, keepdims=True)
    i2 = jnp.min(jnp.where(d2 == m2, iota, BIGI), axis=1, keepdims=True)
    d2 = jnp.where(iota == i2, BIGD, d2)
    m3 = jnp.min(d2, axis=1, keepdims=True)
    i3 = jnp.min(jnp.where(d2 == m3, iota, BIGI), axis=1, keepdims=True)

    dist = jnp.sqrt(jnp.concatenate([m1, m2, m3], axis=1))
    dist = jnp.maximum(dist, 1e-10)
    w = 1.0 / dist
    wn = w / (jnp.sum(w, axis=1, keepdims=True) + 1e-16)
    idxc = jnp.concatenate(
        [i1, i2, i3, jnp.zeros((BY, 5), jnp.int32)], axis=1)
    wc = jnp.concatenate(
        [wn, jnp.zeros((BY, 5), jnp.float32)], axis=1)
    idx_ref[...] = jnp.transpose(idxc, (1, 0)).reshape(1, 8, 128)
    w_ref[...] = jnp.transpose(wc, (1, 0)).reshape(1, 8, 128)


@functools.lru_cache(maxsize=4)
def _make_knn(nblocks, block_off):
    return pl.pallas_call(
        _knn_body,
        grid=(nblocks,),
        in_specs=[
            pl.BlockSpec((128, 3), lambda i: (i + block_off, 0)),
            pl.BlockSpec((3, NIN_PAD), lambda i: (0, 0)),
        ],
        out_specs=[
            pl.BlockSpec((1, 8, 128), lambda i: (i, 0, 0)),
            pl.BlockSpec((1, 8, 128), lambda i: (i, 0, 0)),
        ],
        out_shape=[
            jax.ShapeDtypeStruct((nblocks, 8, 128), jnp.int32),
            jax.ShapeDtypeStruct((nblocks, 8, 128), jnp.float32),
        ],
    )


def _sc_interp_body(nchunk, in_x_hbm, idx3d_hbm, w3d_hbm, out_hbm,
                    idx_v, w_v, rows_v, out_v, sem):
    nc = plsc.get_sparse_core_info().num_cores
    wid = lax.axis_index("s") * nc + lax.axis_index("c")
    pltpu.sync_copy(idx3d_hbm.at[wid], idx_v)
    pltpu.sync_copy(w3d_hbm.at[wid], w_v)

    def chunk_body(j, _):
        handles = [
            pltpu.async_copy(
                in_x_hbm.at[idx_v.at[8 * j + s]],
                rows_v.at[pl.ds(s * 128, 128)],
                sem,
            )
            for s in range(KNN_K)
        ]
        for h in handles:
            h.wait()

        def grp_body(q, _):
            base = q * 16
            wv0 = w_v[8 * j, pl.ds(base, 16)]
            wv1 = w_v[8 * j + 1, pl.ds(base, 16)]
            wv2 = w_v[8 * j + 2, pl.ds(base, 16)]
            for t in range(16):
                p = base + t
                w0, w1, w2 = wv0[t], wv1[t], wv2[t]
                for c in range(D // 16):
                    sl = pl.ds(c * 16, 16)
                    acc = rows_v[p, sl] * w0
                    acc = acc + rows_v[128 + p, sl] * w1
                    acc = acc + rows_v[256 + p, sl] * w2
                    out_v[p, sl] = acc
            return 0

        lax.fori_loop(0, CHUNK // 16, grp_body, 0)
        pltpu.sync_copy(
            out_v,
            out_hbm.at[pl.ds((wid * nchunk + j) * CHUNK, CHUNK)])
        return 0

    lax.fori_loop(0, nchunk, chunk_body, 0)


@functools.lru_cache(maxsize=4)
def _make_sc_interp(nblocks):
    nchunk = nblocks // NW
    return functools.partial(
        pl.kernel,
        mesh=plsc.VectorSubcoreMesh(core_axis_name="c", subcore_axis_name="s"),
        out_type=jax.ShapeDtypeStruct((nblocks * 128, D), jnp.float32),
        scratch_types=[
            pltpu.VMEM((nchunk * 8, 128), jnp.int32),
            pltpu.VMEM((nchunk * 8, 128), jnp.float32),
            pltpu.VMEM((CHUNK * KNN_K, D), jnp.float32),
            pltpu.VMEM((CHUNK, D), jnp.float32),
            pltpu.SemaphoreType.DMA,
        ],
    )(functools.partial(_sc_interp_body, nchunk))


def _concat_body(aggr_a_ref, aggr_b_ref, skip_x_ref, skip_pos_ref, out_ref):
    i = pl.program_id(0)

    @pl.when(i < S1B)
    def _():
        out_ref[:, 0:D] = aggr_a_ref[...]

    @pl.when(i >= S1B)
    def _():
        out_ref[:, 0:D] = aggr_b_ref[...]

    out_ref[:, D:2 * D] = skip_x_ref[...]
    out_ref[:, 2 * D:2 * D + 3] = skip_pos_ref[...]


_concat_call = pl.pallas_call(
    _concat_body,
    grid=(NB_OUT,),
    in_specs=[
        pl.BlockSpec((128, D), lambda i: (jnp.minimum(i, S1B - 1), 0)),
        pl.BlockSpec((128, D), lambda i: (jnp.clip(i - S1B, 0, S2B - 1), 0)),
        pl.BlockSpec((128, D), lambda i: (i, 0)),
        pl.BlockSpec((128, 3), lambda i: (i, 0)),
    ],
    out_specs=pl.BlockSpec((128, 2 * D + 3), lambda i: (i, 0)),
    out_shape=jax.ShapeDtypeStruct((N_SKIP, 2 * D + 3), jnp.float32),
)


def _concat_single_body(aggr_ref, skip_x_ref, skip_pos_ref, out_ref):
    out_ref[:, 0:D] = aggr_ref[...]
    out_ref[:, D:2 * D] = skip_x_ref[...]
    out_ref[:, 2 * D:2 * D + 3] = skip_pos_ref[...]


_concat_single_call = pl.pallas_call(
    _concat_single_body,
    grid=(NB_OUT,),
    in_specs=[
        pl.BlockSpec((128, D), lambda i: (i, 0)),
        pl.BlockSpec((128, D), lambda i: (i, 0)),
        pl.BlockSpec((128, 3), lambda i: (i, 0)),
    ],
    out_specs=pl.BlockSpec((128, 2 * D + 3), lambda i: (i, 0)),
    out_shape=jax.ShapeDtypeStruct((N_SKIP, 2 * D + 3), jnp.float32),
)


def kernel(in_x, in_pos, in_batch, skip_x, skip_pos, skip_batch):
    del in_batch
    skip_pos_p = jnp.pad(skip_pos, ((0, NB * 128 - N_SKIP), (0, 0)))
    in_pos_t = jnp.pad(in_pos.T, ((0, 0), (0, NIN_PAD - N_IN)),
                       constant_values=1e9)
    idx8, w8 = _make_knn(NB, 0)(skip_pos_p, in_pos_t)
    aggr = _make_sc_interp(NB)(
        in_x, idx8.reshape(NW, -1, 128), w8.reshape(NW, -1, 128))
    x1 = _concat_single_call(aggr, skip_x, skip_pos)
    return (x1, skip_pos, skip_batch)


# speedup vs baseline: 1.0473x; 1.0473x over previous
import functools

import jax
import jax.numpy as jnp
from jax import lax
from jax.experimental import pallas as pl
from jax.experimental.pallas import tpu as pltpu
from jax.experimental.pallas import tpu_sc as plsc

KNN_K = 3
N_IN = 12500
N_SKIP = 50000
D = 128

NIN_PAD = 12544
NW = 32
NB = 416
S1B = 256
S2B = 160
NB_OUT = 391
CHUNK = 128


BY = 128


def _knn_body(skip_pos_ref, in_pos_t_ref, idx_ref, w_ref):
    BIGI = jnp.int32(2**30)
    BIGD = jnp.float32(1e30)
    y = skip_pos_ref[...]
    t0 = y[:, 0:1] - in_pos_t_ref[0:1, :]
    t1 = y[:, 1:2] - in_pos_t_ref[1:2, :]
    t2 = y[:, 2:3] - in_pos_t_ref[2:3, :]
    d2 = (t0 * t0 + t1 * t1) + t2 * t2
    iota = lax.broadcasted_iota(jnp.int32, (BY, NIN_PAD), 1)

    m1 = jnp.min(d2, axis=1, keepdims=True)
    i1 = jnp.min(jnp.where(d2 == m1, iota, BIGI), axis=1, keepdims=True)
    d2 = jnp.where(iota == i1, BIGD, d2)
    m2 = jnp.min(d2, axis=1, keepdims=True)
    i2 = jnp.min(jnp.where(d2 == m2, iota, BIGI), axis=1, keepdims=True)
    d2 = jnp.where(iota == i2, BIGD, d2)
    m3 = jnp.min(d2, axis=1, keepdims=True)
    i3 = jnp.min(jnp.where(d2 == m3, iota, BIGI), axis=1, keepdims=True)

    dist = jnp.sqrt(jnp.concatenate([m1, m2, m3], axis=1))
    dist = jnp.maximum(dist, 1e-10)
    w = 1.0 / dist
    wn = w / (jnp.sum(w, axis=1, keepdims=True) + 1e-16)
    idxc = jnp.concatenate(
        [i1, i2, i3, jnp.zeros((BY, 5), jnp.int32)], axis=1)
    wc = jnp.concatenate(
        [wn, jnp.zeros((BY, 5), jnp.float32)], axis=1)
    for h in range(BY // 128):
        idx_ref[h] = jnp.transpose(idxc[h * 128:(h + 1) * 128], (1, 0))
        w_ref[h] = jnp.transpose(wc[h * 128:(h + 1) * 128], (1, 0))


@functools.lru_cache(maxsize=4)
def _make_knn(nblocks, block_off):
    hb = BY // 128
    return pl.pallas_call(
        _knn_body,
        grid=(nblocks // hb,),
        in_specs=[
            pl.BlockSpec((BY, 3), lambda i: (i + block_off // hb, 0)),
            pl.BlockSpec((3, NIN_PAD), lambda i: (0, 0)),
        ],
        out_specs=[
            pl.BlockSpec((hb, 8, 128), lambda i: (i, 0, 0)),
            pl.BlockSpec((hb, 8, 128), lambda i: (i, 0, 0)),
        ],
        out_shape=[
            jax.ShapeDtypeStruct((nblocks, 8, 128), jnp.int32),
            jax.ShapeDtypeStruct((nblocks, 8, 128), jnp.float32),
        ],
    )


def _sc_interp_body(nchunk, in_x_hbm, idx3d_hbm, w3d_hbm, out_hbm,
                    idx_v, w_v, rows_v, out_v, sem):
    nc = plsc.get_sparse_core_info().num_cores
    wid = lax.axis_index("s") * nc + lax.axis_index("c")
    pltpu.sync_copy(idx3d_hbm.at[wid], idx_v)
    pltpu.sync_copy(w3d_hbm.at[wid], w_v)

    def chunk_body(j, _):
        handles = [
            pltpu.async_copy(
                in_x_hbm.at[idx_v.at[8 * j + s]],
                rows_v.at[pl.ds(s * 128, 128)],
                sem,
            )
            for s in range(KNN_K)
        ]
        for h in handles:
            h.wait()

        def grp_body(q, _):
            base = q * 16
            wv0 = w_v[8 * j, pl.ds(base, 16)]
            wv1 = w_v[8 * j + 1, pl.ds(base, 16)]
            wv2 = w_v[8 * j + 2, pl.ds(base, 16)]
            for t in range(16):
                p = base + t
                w0, w1, w2 = wv0[t], wv1[t], wv2[t]
                for c in range(D // 16):
                    sl = pl.ds(c * 16, 16)
                    acc = rows_v[p, sl] * w0
                    acc = acc + rows_v[128 + p, sl] * w1
                    acc = acc + rows_v[256 + p, sl] * w2
                    out_v[p, sl] = acc
            return 0

        lax.fori_loop(0, CHUNK // 16, grp_body, 0)
        pltpu.sync_copy(
            out_v,
            out_hbm.at[pl.ds((wid * nchunk + j) * CHUNK, CHUNK)])
        return 0

    lax.fori_loop(0, nchunk, chunk_body, 0)


@functools.lru_cache(maxsize=4)
def _make_sc_interp(nblocks):
    nchunk = nblocks // NW
    return functools.partial(
        pl.kernel,
        mesh=plsc.VectorSubcoreMesh(core_axis_name="c", subcore_axis_name="s"),
        out_type=jax.ShapeDtypeStruct((nblocks * 128, D), jnp.float32),
        scratch_types=[
            pltpu.VMEM((nchunk * 8, 128), jnp.int32),
            pltpu.VMEM((nchunk * 8, 128), jnp.float32),
            pltpu.VMEM((CHUNK * KNN_K, D), jnp.float32),
            pltpu.VMEM((CHUNK, D), jnp.float32),
            pltpu.SemaphoreType.DMA,
        ],
    )(functools.partial(_sc_interp_body, nchunk))


def _concat_body(aggr_a_ref, aggr_b_ref, skip_x_ref, skip_pos_ref, out_ref):
    i = pl.program_id(0)

    @pl.when(i < S1B)
    def _():
        out_ref[:, 0:D] = aggr_a_ref[...]

    @pl.when(i >= S1B)
    def _():
        out_ref[:, 0:D] = aggr_b_ref[...]

    out_ref[:, D:2 * D] = skip_x_ref[...]
    out_ref[:, 2 * D:2 * D + 3] = skip_pos_ref[...]


_concat_call = pl.pallas_call(
    _concat_body,
    grid=(NB_OUT,),
    in_specs=[
        pl.BlockSpec((128, D), lambda i: (jnp.minimum(i, S1B - 1), 0)),
        pl.BlockSpec((128, D), lambda i: (jnp.clip(i - S1B, 0, S2B - 1), 0)),
        pl.BlockSpec((128, D), lambda i: (i, 0)),
        pl.BlockSpec((128, 3), lambda i: (i, 0)),
    ],
    out_specs=pl.BlockSpec((128, 2 * D + 3), lambda i: (i, 0)),
    out_shape=jax.ShapeDtypeStruct((N_SKIP, 2 * D + 3), jnp.float32),
)


def _concat_single_body(aggr_ref, skip_x_ref, skip_pos_ref, out_ref):
    out_ref[:, 0:D] = aggr_ref[...]
    out_ref[:, D:2 * D] = skip_x_ref[...]
    out_ref[:, 2 * D:2 * D + 3] = skip_pos_ref[...]


_concat_single_call = pl.pallas_call(
    _concat_single_body,
    grid=(NB_OUT,),
    in_specs=[
        pl.BlockSpec((128, D), lambda i: (i, 0)),
        pl.BlockSpec((128, D), lambda i: (i, 0)),
        pl.BlockSpec((128, 3), lambda i: (i, 0)),
    ],
    out_specs=pl.BlockSpec((128, 2 * D + 3), lambda i: (i, 0)),
    out_shape=jax.ShapeDtypeStruct((N_SKIP, 2 * D + 3), jnp.float32),
)


def kernel(in_x, in_pos, in_batch, skip_x, skip_pos, skip_batch):
    del in_batch
    skip_pos_p = jnp.pad(skip_pos, ((0, NB * 128 - N_SKIP), (0, 0)))
    in_pos_t = jnp.pad(in_pos.T, ((0, 0), (0, NIN_PAD - N_IN)),
                       constant_values=1e9)
    idx8a, w8a = _make_knn(S1B, 0)(skip_pos_p, in_pos_t)
    idx8b, w8b = _make_knn(S2B, S1B)(skip_pos_p, in_pos_t)
    aggr_a = _make_sc_interp(S1B)(
        in_x, idx8a.reshape(NW, -1, 128), w8a.reshape(NW, -1, 128))
    aggr_b = _make_sc_interp(S2B)(
        in_x, idx8b.reshape(NW, -1, 128), w8b.reshape(NW, -1, 128))
    x1 = _concat_call(aggr_a, aggr_b, skip_x, skip_pos)
    return (x1, skip_pos, skip_batch)
